# baseline (speedup 1.0000x reference)
import functools

import jax
import jax.numpy as jnp
from jax import lax
from jax.experimental import pallas as pl
from jax.experimental.pallas import tpu as pltpu
from jax.experimental.pallas import tpu_sc as plsc


def _sc_edge_call(ev_sc, adjs_sc, vecb, n_tiles):
    n_workers = 32
    per_w = n_tiles // n_workers
    mesh = plsc.VectorSubcoreMesh(core_axis_name="c", subcore_axis_name="s")

    @functools.partial(
        pl.kernel, mesh=mesh,
        out_type=jax.ShapeDtypeStruct((n_tiles, 8, 128), jnp.float32),
        scratch_types=[
            pltpu.VMEM((8, 4, 128), jnp.float32),
            pltpu.VMEM((8, 128), jnp.float32),
            pltpu.VMEM((8, 128), jnp.float32),
            pltpu.VMEM((64,), jnp.float32),
        ],
    )
    def sc_kernel(ev_hbm, adjs_hbm, vecb_hbm, out_hbm, ebuf, mbuf, obuf, vbuf):
        c = lax.axis_index("c")
        s = lax.axis_index("s")
        wid = s * 2 + c
        pltpu.sync_copy(vecb_hbm, vbuf)
        v = [vbuf[pl.ds(e * 16, 16)] for e in range(4)]
        neg = jnp.full((16,), -1e9, jnp.float32)

        def body(t, carry):
            tile = wid * per_w + t
            bi = (tile // 2) * 8
            jb = tile % 2
            pltpu.sync_copy(ev_hbm.at[pl.ds(bi, 8), pl.ds(jb * 4, 4), :], ebuf)
            pltpu.sync_copy(adjs_hbm.at[tile], mbuf)
            for r in range(8):
                for g in range(8):
                    sl = pl.ds(g * 16, 16)
                    acc = (ebuf[r, 0, sl] * v[0] + ebuf[r, 1, sl] * v[1]
                           + ebuf[r, 2, sl] * v[2] + ebuf[r, 3, sl] * v[3])
                    m = mbuf[r, sl] > 0.5
                    obuf[r, sl] = jnp.where(m, acc, neg)
            pltpu.sync_copy(obuf, out_hbm.at[tile])
            return carry

        lax.fori_loop(0, per_w, body, 0)

    return sc_kernel(ev_sc, adjs_sc, vecb)


def _tc_body(atoms_ref, am_ref, w_ref, asrc_ref, adst_ref, bias_ref, out_ref,
             *, n):
    x = atoms_ref[0]
    xl = jax.lax.dot_general(x, w_ref[...], (((1,), (0,)), ((), ())),
                             preferred_element_type=jnp.float32)
    a_src = jax.lax.dot_general(xl, asrc_ref[...], (((1,), (1,)), ((), ())),
                                preferred_element_type=jnp.float32)
    a_dst = jax.lax.dot_general(adst_ref[...], xl, (((1,), (1,)), ((), ())),
                                preferred_element_type=jnp.float32)
    am4 = am_ref[0].reshape(n // 8, 2, 8, 128)
    for h in range(2):
        am_h = am4[:, h, :, :].reshape(n, 128)
        lg = a_src + a_dst[:, h * 128:(h + 1) * 128] + am_h
        lg = jnp.maximum(lg, 0.2 * lg)
        mx = jnp.max(lg, axis=0, keepdims=True)
        ex = jnp.exp(lg - mx)
        sinv = 1.0 / jnp.sum(ex, axis=0, keepdims=True)
        att = ex * sinv
        att = jnp.where(mx > -1e7, att, 0.0)
        out = jax.lax.dot_general(att, xl, (((0,), (0,)), ((), ())),
                                  preferred_element_type=jnp.float32)
        out_ref[0, h * 128:(h + 1) * 128, :] = out + bias_ref[...]


def kernel(atoms, adjs, edges, W, att_src, att_dst, W_edge, att_edge, bias):
    B, N, D = atoms.shape
    E = edges.shape[-1]
    C = W.shape[-1]
    ev_sc = edges.reshape(B, N, 2, 128, E).transpose(0, 1, 2, 4, 3)
    ev_sc = ev_sc.reshape(B * N, 2 * E, 128)
    adjs_sc = adjs.reshape(B, N // 8, 8, 2, 128).transpose(0, 1, 3, 2, 4)
    adjs_sc = adjs_sc.reshape(B * (N // 8) * 2, 8, 128)
    vec = jnp.sum(W_edge.reshape(E, C) * att_edge.reshape(1, C), axis=1)
    vecb = jnp.repeat(vec, 16)

    n_tiles = B * (N // 8) * 2
    am = _sc_edge_call(ev_sc, adjs_sc, vecb, n_tiles)
    am_v = am.reshape(B, N * 2, 128)

    w2 = W.reshape(D, C)
    bias2 = bias.reshape(1, C)
    body = functools.partial(_tc_body, n=N)
    out = pl.pallas_call(
        body,
        grid=(B,),
        in_specs=[
            pl.BlockSpec((1, N, D), lambda b: (b, 0, 0)),
            pl.BlockSpec((1, N * 2, 128), lambda b: (b, 0, 0)),
            pl.BlockSpec((D, C), lambda b: (0, 0)),
            pl.BlockSpec((1, C), lambda b: (0, 0)),
            pl.BlockSpec((1, C), lambda b: (0, 0)),
            pl.BlockSpec((1, C), lambda b: (0, 0)),
        ],
        out_specs=pl.BlockSpec((1, N, C), lambda b: (b, 0, 0)),
        out_shape=jax.ShapeDtypeStruct((B, N, C), jnp.float32),
    )(atoms, am_v, w2, att_src, att_dst, bias2)
    return out


# speedup vs baseline: 1.7432x; 1.7432x over previous
import functools

import jax
import jax.numpy as jnp
from jax import lax
from jax.experimental import pallas as pl
from jax.experimental.pallas import tpu as pltpu
from jax.experimental.pallas import tpu_sc as plsc


def _sc_edge_call(ev_sc, adjs_sc, vecb, n_tiles):
    n_workers = 32
    per_w = n_tiles // n_workers
    mesh = plsc.VectorSubcoreMesh(core_axis_name="c", subcore_axis_name="s")

    n_blk = per_w // 2
    n_pairs = n_blk // 2

    @functools.partial(
        pl.kernel, mesh=mesh,
        out_type=jax.ShapeDtypeStruct((n_tiles, 8, 128), jnp.float32),
        scratch_types=[
            pltpu.VMEM((8, 8, 128), jnp.float32),
            pltpu.VMEM((8, 8, 128), jnp.float32),
            pltpu.VMEM((2, 8, 128), jnp.float32),
            pltpu.VMEM((2, 8, 128), jnp.float32),
            pltpu.VMEM((2, 8, 128), jnp.float32),
            pltpu.VMEM((64,), jnp.float32),
            pltpu.SemaphoreType.DMA,
            pltpu.SemaphoreType.DMA,
            pltpu.SemaphoreType.DMA,
            pltpu.SemaphoreType.DMA,
        ],
    )
    def sc_kernel(ev_hbm, adjs_hbm, vecb_hbm, out_hbm,
                  ebuf_a, ebuf_b, mbuf_a, mbuf_b, obuf, vbuf,
                  esem_a, esem_b, msem_a, msem_b):
        c = lax.axis_index("c")
        s = lax.axis_index("s")
        wid = s * 2 + c
        blk0 = wid * n_blk
        pltpu.sync_copy(vecb_hbm, vbuf)
        v = [vbuf[pl.ds(e * 16, 16)] for e in range(4)]
        neg = jnp.full((16,), -1e9, jnp.float32)

        def start_in(blk, ebuf, mbuf, esem, msem):
            pltpu.async_copy(ev_hbm.at[pl.ds(blk * 8, 8)], ebuf, esem)
            pltpu.async_copy(adjs_hbm.at[pl.ds(blk * 2, 2)], mbuf, msem)

        def wait_in(blk, ebuf, mbuf, esem, msem):
            pltpu.make_async_copy(ev_hbm.at[pl.ds(blk * 8, 8)], ebuf, esem).wait()
            pltpu.make_async_copy(adjs_hbm.at[pl.ds(blk * 2, 2)], mbuf, msem).wait()

        def compute_out(blk, ebuf, mbuf):
            for jb in range(2):
                for r in range(8):
                    for g in range(8):
                        sl = pl.ds(g * 16, 16)
                        acc = (ebuf[r, jb * 4 + 0, sl] * v[0]
                               + ebuf[r, jb * 4 + 1, sl] * v[1]
                               + ebuf[r, jb * 4 + 2, sl] * v[2]
                               + ebuf[r, jb * 4 + 3, sl] * v[3])
                        m = mbuf[jb, r, sl] > 0.5
                        obuf[jb, r, sl] = jnp.where(m, acc, neg)
            pltpu.sync_copy(obuf, out_hbm.at[pl.ds(blk * 2, 2)])

        start_in(blk0, ebuf_a, mbuf_a, esem_a, msem_a)

        def body(k, carry):
            blk_a = blk0 + 2 * k
            blk_b = blk_a + 1
            start_in(blk_b, ebuf_b, mbuf_b, esem_b, msem_b)
            wait_in(blk_a, ebuf_a, mbuf_a, esem_a, msem_a)
            compute_out(blk_a, ebuf_a, mbuf_a)

            @pl.when(k < n_pairs - 1)
            def _():
                start_in(blk_a + 2, ebuf_a, mbuf_a, esem_a, msem_a)

            wait_in(blk_b, ebuf_b, mbuf_b, esem_b, msem_b)
            compute_out(blk_b, ebuf_b, mbuf_b)
            return carry

        lax.fori_loop(0, n_pairs, body, 0)

    return sc_kernel(ev_sc, adjs_sc, vecb)


def _tc_body(atoms_ref, am_ref, w_ref, asrc_ref, adst_ref, bias_ref, out_ref,
             *, n):
    x = atoms_ref[0]
    xl = jax.lax.dot_general(x, w_ref[...], (((1,), (0,)), ((), ())),
                             preferred_element_type=jnp.float32)
    a_src = jax.lax.dot_general(xl, asrc_ref[...], (((1,), (1,)), ((), ())),
                                preferred_element_type=jnp.float32)
    a_dst = jax.lax.dot_general(adst_ref[...], xl, (((1,), (1,)), ((), ())),
                                preferred_element_type=jnp.float32)
    am4 = am_ref[0].reshape(n // 8, 2, 8, 128)
    for h in range(2):
        am_h = am4[:, h, :, :].reshape(n, 128)
        lg = a_src + a_dst[:, h * 128:(h + 1) * 128] + am_h
        lg = jnp.maximum(lg, 0.2 * lg)
        mx = jnp.max(lg, axis=0, keepdims=True)
        ex = jnp.exp(lg - mx)
        sinv = 1.0 / jnp.sum(ex, axis=0, keepdims=True)
        att = ex * sinv
        att = jnp.where(mx > -1e7, att, 0.0)
        out = jax.lax.dot_general(att, xl, (((0,), (0,)), ((), ())),
                                  preferred_element_type=jnp.float32)
        out_ref[0, h * 128:(h + 1) * 128, :] = out + bias_ref[...]


def kernel(atoms, adjs, edges, W, att_src, att_dst, W_edge, att_edge, bias):
    B, N, D = atoms.shape
    E = edges.shape[-1]
    C = W.shape[-1]
    ev_sc = edges.reshape(B, N, 2, 128, E).transpose(0, 1, 2, 4, 3)
    ev_sc = ev_sc.reshape(B * N, 2 * E, 128)
    adjs_sc = adjs.reshape(B, N // 8, 8, 2, 128).transpose(0, 1, 3, 2, 4)
    adjs_sc = adjs_sc.reshape(B * (N // 8) * 2, 8, 128)
    vec = jnp.sum(W_edge.reshape(E, C) * att_edge.reshape(1, C), axis=1)
    vecb = jnp.repeat(vec, 16)

    n_tiles = B * (N // 8) * 2
    am = _sc_edge_call(ev_sc, adjs_sc, vecb, n_tiles)
    am_v = am.reshape(B, N * 2, 128)

    w2 = W.reshape(D, C)
    bias2 = bias.reshape(1, C)
    body = functools.partial(_tc_body, n=N)
    out = pl.pallas_call(
        body,
        grid=(B,),
        in_specs=[
            pl.BlockSpec((1, N, D), lambda b: (b, 0, 0)),
            pl.BlockSpec((1, N * 2, 128), lambda b: (b, 0, 0)),
            pl.BlockSpec((D, C), lambda b: (0, 0)),
            pl.BlockSpec((1, C), lambda b: (0, 0)),
            pl.BlockSpec((1, C), lambda b: (0, 0)),
            pl.BlockSpec((1, C), lambda b: (0, 0)),
        ],
        out_specs=pl.BlockSpec((1, N, C), lambda b: (b, 0, 0)),
        out_shape=jax.ShapeDtypeStruct((B, N, C), jnp.float32),
    )(atoms, am_v, w2, att_src, att_dst, bias2)
    return out


# speedup vs baseline: 3.5280x; 2.0239x over previous
import functools

import jax
import jax.numpy as jnp
from jax.experimental import pallas as pl
from jax.experimental.pallas import tpu as pltpu


def _gat_body(atoms_ref, adjs_ref, ev_ref, w_ref, asrc_ref, adst_ref,
              wedge_ref, aedge_ref, bias_ref, out_ref, p_ref, *, n, e):
    b = pl.program_id(0)
    rows = n * 2 * e

    @pl.when(b == 0)
    def _build_p():
        vec = jnp.sum(wedge_ref[...] * aedge_ref[...], axis=1, keepdims=True)
        r = jax.lax.broadcasted_iota(jnp.int32, (rows, 2 * n), 0)
        c = jax.lax.broadcasted_iota(jnp.int32, (rows, 2 * n), 1)
        cond = ((r // (2 * e)) == (c % n)) & (((r // e) % 2) == (c // n))
        m = jnp.zeros((rows, 2 * n), jnp.float32)
        for d in range(e):
            m = m + jnp.where(cond & ((r % e) == d), vec[d:d + 1, 0:1], 0.0)
        p_ref[...] = m.astype(jnp.bfloat16)

    x = atoms_ref[0]
    xl = jax.lax.dot_general(x, w_ref[...], (((1,), (0,)), ((), ())),
                             preferred_element_type=jnp.float32)
    a_src = jax.lax.dot_general(xl, asrc_ref[...], (((1,), (1,)), ((), ())),
                                preferred_element_type=jnp.float32)
    a_dst = jax.lax.dot_general(adst_ref[...], xl, (((1,), (1,)), ((), ())),
                                preferred_element_type=jnp.float32)
    ae5 = jax.lax.dot_general(p_ref[...], ev_ref[...].astype(jnp.bfloat16),
                              (((0,), (0,)), ((), ())),
                              preferred_element_type=jnp.float32)
    for h in range(2):
        ae_h = ae5[h * n:(h + 1) * n, :]
        lg = a_src + a_dst[:, h * 128:(h + 1) * 128] + ae_h
        lg = jnp.maximum(lg, 0.2 * lg)
        mask = adjs_ref[0][:, h * 128:(h + 1) * 128] > 0.5
        ml = jnp.where(mask, lg, -1e9)
        mx = jnp.max(ml, axis=0, keepdims=True)
        ex = jnp.exp(ml - mx)
        sinv = 1.0 / jnp.sum(ex, axis=0, keepdims=True)
        att = jnp.where(mask, ex * sinv, 0.0)
        out = jax.lax.dot_general(att, xl, (((0,), (0,)), ((), ())),
                                  preferred_element_type=jnp.float32)
        out_ref[0, h * 128:(h + 1) * 128, :] = out + bias_ref[...]


def kernel(atoms, adjs, edges, W, att_src, att_dst, W_edge, att_edge, bias):
    B, N, D = atoms.shape
    E = edges.shape[-1]
    C = W.shape[-1]
    ev = edges.reshape(B, N, 2, 128, E).transpose(0, 1, 2, 4, 3)
    ev = ev.reshape(B * N * 2 * E, 128)
    rows = N * 2 * E
    w2 = W.reshape(D, C)
    wedge = W_edge.reshape(E, C)
    bias2 = bias.reshape(1, C)

    body = functools.partial(_gat_body, n=N, e=E)
    out = pl.pallas_call(
        body,
        grid=(B,),
        in_specs=[
            pl.BlockSpec((1, N, D), lambda b: (b, 0, 0)),
            pl.BlockSpec((1, N, N), lambda b: (b, 0, 0)),
            pl.BlockSpec((rows, 128), lambda b: (b, 0)),
            pl.BlockSpec((D, C), lambda b: (0, 0)),
            pl.BlockSpec((1, C), lambda b: (0, 0)),
            pl.BlockSpec((1, C), lambda b: (0, 0)),
            pl.BlockSpec((E, C), lambda b: (0, 0)),
            pl.BlockSpec((1, C), lambda b: (0, 0)),
            pl.BlockSpec((1, C), lambda b: (0, 0)),
        ],
        out_specs=pl.BlockSpec((1, N, C), lambda b: (b, 0, 0)),
        out_shape=jax.ShapeDtypeStruct((B, N, C), jnp.float32),
        scratch_shapes=[pltpu.VMEM((rows, 2 * N), jnp.bfloat16)],
    )(atoms, adjs, ev, w2, att_src, att_dst, wedge, att_edge, bias2)
    return out
